# baseline (speedup 1.0000x reference)
import jax
import jax.numpy as jnp
from jax.experimental import pallas as pl
from jax.experimental.pallas import tpu as pltpu

N = 4096
D_IN = 64
D_OUT = 64
BM = 512
NB = N // BM
H = N // 2


def _strip_dot(a_s, gs_s, r0, r1, lo, hi):
    return jnp.dot(
        a_s[pl.ds(r0 * BM, (r1 - r0) * BM), pl.ds(lo, hi - lo)],
        gs_s[pl.ds(lo, hi - lo), :],
        preferred_element_type=jnp.float32,
    )


def _fused_kernel(a_ref, f_ref, w_ref, o_ref, a_s, d_s, g_s, gs_s, acc_s):
    j = pl.program_id(0)

    @pl.when(j == 0)
    def _():
        g_s[...] = jnp.dot(
            f_ref[...], w_ref[...].T, preferred_element_type=jnp.float32
        )

    a = a_ref[...]
    s = jnp.sum(a, axis=1, keepdims=True)
    inv = jax.lax.rsqrt(s)
    d_j = jnp.where(jnp.isinf(inv), 0.0, inv)
    d_s[pl.ds(j * BM, BM), :] = d_j
    a_s[pl.ds(j * BM, BM), :] = a.astype(jnp.bfloat16)
    gs_s[pl.ds(j * BM, BM), :] = (d_j * g_s[pl.ds(j * BM, BM), :]).astype(
        jnp.bfloat16
    )

    @pl.when(j == 4)
    def _():
        acc_s[pl.ds(0, 4 * BM), :] = _strip_dot(a_s, gs_s, 0, 4, 0, H)

    @pl.when(j == 5)
    def _():
        acc_s[pl.ds(4 * BM, 2 * BM), :] = _strip_dot(a_s, gs_s, 4, 6, 0, H)

    @pl.when(j == 6)
    def _():
        acc_s[pl.ds(6 * BM, BM), :] = _strip_dot(a_s, gs_s, 6, 7, 0, H)

    @pl.when(j == NB - 1)
    def _():
        acc_s[pl.ds(0, 7 * BM), :] += _strip_dot(a_s, gs_s, 0, 7, H, N)
        acc_s[pl.ds(7 * BM, BM), :] = _strip_dot(a_s, gs_s, 7, 8, 0, N)
        o_ref[...] = d_s[...] * acc_s[...]


@jax.jit
def kernel(adj_matrix, feature_matrix, W):
    return pl.pallas_call(
        _fused_kernel,
        grid=(NB,),
        in_specs=[
            pl.BlockSpec((BM, N), lambda i: (i, 0)),
            pl.BlockSpec((N, D_IN), lambda i: (0, 0)),
            pl.BlockSpec((D_OUT, D_IN), lambda i: (0, 0)),
        ],
        out_specs=pl.BlockSpec((N, D_OUT), lambda i: (0, 0)),
        out_shape=jax.ShapeDtypeStruct((N, D_OUT), jnp.float32),
        scratch_shapes=[
            pltpu.VMEM((N, N), jnp.bfloat16),
            pltpu.VMEM((N, 1), jnp.float32),
            pltpu.VMEM((N, D_OUT), jnp.float32),
            pltpu.VMEM((N, D_OUT), jnp.bfloat16),
            pltpu.VMEM((N, D_OUT), jnp.float32),
        ],
        compiler_params=pltpu.CompilerParams(
            dimension_semantics=("arbitrary",),
            vmem_limit_bytes=67000000,
        ),
    )(adj_matrix, feature_matrix, W)


# speedup vs baseline: 1.0581x; 1.0581x over previous
import jax
import jax.numpy as jnp
from jax.experimental import pallas as pl
from jax.experimental.pallas import tpu as pltpu

N = 4096
D_IN = 64
D_OUT = 64
BM = 512
NB = N // BM
H = N // 2


def _strip_dot(a_s, gs_s, r0, r1, lo, hi):
    return jnp.dot(
        a_s[pl.ds(r0 * BM, (r1 - r0) * BM), pl.ds(lo, hi - lo)],
        gs_s[pl.ds(lo, hi - lo), :],
        preferred_element_type=jnp.float32,
    )


def _fused_kernel(a_ref, f_ref, w_ref, o_ref, a_s, d_s, g_s, gs_s, acc_s):
    j = pl.program_id(0)

    @pl.when(j == 0)
    def _():
        g_s[...] = jnp.dot(
            f_ref[...], w_ref[...].T, preferred_element_type=jnp.float32
        )

    a = a_ref[...]
    s = jnp.sum(a, axis=1, keepdims=True)
    inv = jax.lax.rsqrt(s)
    d_j = jnp.where(jnp.isinf(inv), 0.0, inv)
    d_s[pl.ds(j * BM, BM), :] = d_j
    a_s[pl.ds(j * BM, BM), :] = a.astype(jnp.bfloat16)
    gs_s[pl.ds(j * BM, BM), :] = (d_j * g_s[pl.ds(j * BM, BM), :]).astype(
        jnp.bfloat16
    )

    @pl.when(j == 4)
    def _():
        acc_s[pl.ds(0, 2048), :] = _strip_dot(a_s, gs_s, 0, 4, 0, 2048)

    @pl.when(j == 5)
    def _():
        acc_s[pl.ds(2048, 1024), :] = _strip_dot(a_s, gs_s, 4, 6, 0, 2048)
        acc_s[pl.ds(0, 2048), :] += _strip_dot(a_s, gs_s, 0, 4, 2048, 3072)

    @pl.when(j == 6)
    def _():
        acc_s[pl.ds(3072, 512), :] = _strip_dot(a_s, gs_s, 6, 7, 0, 3072)
        acc_s[pl.ds(2048, 1024), :] += _strip_dot(a_s, gs_s, 4, 6, 2048, 3072)
        acc_s[pl.ds(0, 3072), :] += _strip_dot(a_s, gs_s, 0, 6, 3072, 3584)

    @pl.when(j == NB - 1)
    def _():
        acc_s[pl.ds(0, 3072), :] += _strip_dot(a_s, gs_s, 0, 6, 3584, N)
        acc_s[pl.ds(3072, 512), :] += _strip_dot(a_s, gs_s, 6, 7, 3072, N)
        acc_s[pl.ds(3584, 512), :] = _strip_dot(a_s, gs_s, 7, 8, 0, N)
        o_ref[...] = d_s[...] * acc_s[...]


@jax.jit
def kernel(adj_matrix, feature_matrix, W):
    return pl.pallas_call(
        _fused_kernel,
        grid=(NB,),
        in_specs=[
            pl.BlockSpec((BM, N), lambda i: (i, 0)),
            pl.BlockSpec((N, D_IN), lambda i: (0, 0)),
            pl.BlockSpec((D_OUT, D_IN), lambda i: (0, 0)),
        ],
        out_specs=pl.BlockSpec((N, D_OUT), lambda i: (0, 0)),
        out_shape=jax.ShapeDtypeStruct((N, D_OUT), jnp.float32),
        scratch_shapes=[
            pltpu.VMEM((N, N), jnp.bfloat16),
            pltpu.VMEM((N, 1), jnp.float32),
            pltpu.VMEM((N, D_OUT), jnp.float32),
            pltpu.VMEM((N, D_OUT), jnp.bfloat16),
            pltpu.VMEM((N, D_OUT), jnp.float32),
        ],
        compiler_params=pltpu.CompilerParams(
            dimension_semantics=("arbitrary",),
            vmem_limit_bytes=67000000,
        ),
    )(adj_matrix, feature_matrix, W)
